# Initial kernel scaffold; baseline (speedup 1.0000x reference)
import jax
import jax.numpy as jnp
from jax.experimental import pallas as pl


def kernel(x_patient, x_radiomic, x_gene, edge_rp, edge_gp, edge_pp, W_src_rp, W_dst_rp, att_src_rp, att_dst_rp, bias_rp, W_src_gp, W_dst_gp, att_src_gp, att_dst_gp, bias_gp, W_src_pp, W_dst_pp, att_src_pp, att_dst_pp, bias_pp, W_lin, b_lin):
    raise NotImplementedError("write your pallas kernel here")


# speedup vs baseline: 9.2099x; 9.2099x over previous
import functools

import jax
import jax.numpy as jnp
from jax import lax
from jax.experimental import pallas as pl
from jax.experimental.pallas import tpu as pltpu
from jax.experimental.pallas import tpu_sc as plsc

N_NODES = 10000
E_EDGES = 320000
D_IN = 128
H_DIM = 64
OUT_DIM = 32

NW = 32
EDGES_PER_W = E_EDGES // NW
CHUNK = 80
NCHUNK = EDGES_PER_W // CHUNK
GROUPS = CHUNK // 16
ACC_ROWS = 10240

_f32 = jnp.float32
_i32 = jnp.int32


_BP = 1000


def _tc_pre_body(xs_ref, xp_ref, wsrc_ref, wdst_ref, asrc_ref, adst_ref,
                 hs_ref, es_ref, ed_ref):
    x = xs_ref[0]
    hs = jnp.dot(x, wsrc_ref[0], preferred_element_type=_f32)
    hs_ref[0] = hs
    es_ref[0, 0, 0] = jnp.sum(hs * asrc_ref[0, 0][None, :], axis=1)
    hd = jnp.dot(xp_ref[...], wdst_ref[0], preferred_element_type=_f32)
    ed_ref[0, 0, 0] = jnp.sum(hd * adst_ref[0, 0][None, :], axis=1)


def _tc_pre(Xs, Xp, Wsrc, Wdst, Asrc, Adst):
    grid = (3, N_NODES // _BP)
    return pl.pallas_call(
        _tc_pre_body,
        grid=grid,
        in_specs=[
            pl.BlockSpec((1, _BP, D_IN), lambda r, i: (r, i, 0)),
            pl.BlockSpec((_BP, D_IN), lambda r, i: (i, 0)),
            pl.BlockSpec((1, D_IN, H_DIM), lambda r, i: (r, 0, 0)),
            pl.BlockSpec((1, D_IN, H_DIM), lambda r, i: (r, 0, 0)),
            pl.BlockSpec((1, 1, H_DIM), lambda r, i: (r, 0, 0)),
            pl.BlockSpec((1, 1, H_DIM), lambda r, i: (r, 0, 0)),
        ],
        out_specs=[
            pl.BlockSpec((1, _BP, H_DIM), lambda r, i: (r, i, 0)),
            pl.BlockSpec((1, 1, 1, _BP), lambda r, i: (r, i, 0, 0)),
            pl.BlockSpec((1, 1, 1, _BP), lambda r, i: (r, i, 0, 0)),
        ],
        out_shape=[
            jax.ShapeDtypeStruct((3, N_NODES, H_DIM), _f32),
            jax.ShapeDtypeStruct((3, N_NODES // _BP, 1, _BP), _f32),
            jax.ShapeDtypeStruct((3, N_NODES // _BP, 1, _BP), _f32),
        ],
    )(Xs, Xp, Wsrc, Wdst, Asrc, Adst)


_SC_MESH = plsc.VectorSubcoreMesh(core_axis_name="c", subcore_axis_name="s")


def _sc_edge_body(hs0, hs1, hs2, es_h, ed_h, src0, src1, src2,
                  dst0, dst1, dst2,
                  acc_out, den_out,
                  es_v, ed_v, den_v, rows_v, sidx_v, didx_v, zero_v,
                  acc, sem):
    cid = lax.axis_index("c")
    sid = lax.axis_index("s")
    wid = sid * 2 + cid

    def _zrow(j, carry):
        for c4 in range(H_DIM // 16):
            zero_v[j, pl.ds(c4 * 16, 16)] = jnp.zeros((16,), _f32)
        return carry
    lax.fori_loop(0, CHUNK, _zrow, 0)

    stripe = ACC_ROWS // 16
    iota16 = lax.broadcasted_iota(_i32, (16,), 0)

    for r, (hs_h, src_h, dst_h) in enumerate(
            ((hs0, src0, dst0), (hs1, src1, dst1), (hs2, src2, dst2))):
        def _zacc(q, carry):
            pltpu.sync_copy(zero_v, acc.at[pl.ds(sid * stripe + q * CHUNK, CHUNK)])
            return carry
        lax.fori_loop(0, stripe // CHUNK, _zacc, 0)
        plsc.subcore_barrier()

        pltpu.sync_copy(es_h.at[r, 0], es_v)
        pltpu.sync_copy(ed_h.at[r, 0], ed_v)

        def _zden(i, carry):
            den_v[pl.ds(i * 16, 16)] = jnp.zeros((16,), _f32)
            return carry
        lax.fori_loop(0, N_NODES // 16, _zden, 0)

        def _chunk(k, carry):
            base = wid * EDGES_PER_W + k * CHUNK
            pltpu.sync_copy(src_h.at[pl.ds(base, CHUNK)], sidx_v)
            pltpu.sync_copy(dst_h.at[pl.ds(base, CHUNK)], didx_v)
            pltpu.async_copy(hs_h.at[sidx_v], rows_v, sem).wait()
            for g in range(GROUPS):
                sv = sidx_v[pl.ds(g * 16, 16)]
                dv = didx_v[pl.ds(g * 16, 16)]
                e = plsc.load_gather(es_v, [sv]) + plsc.load_gather(ed_v, [dv])
                e = jnp.where(e >= 0.0, e, 0.2 * e)
                ex = jnp.exp(e)
                plsc.addupdate_scatter(den_v, [dv], ex)
                jvec = iota16 + (g * 16)
                for c in range(H_DIM):
                    cvec = jnp.full((16,), c, _i32)
                    t = plsc.load_gather(rows_v, [jvec, cvec])
                    plsc.store_scatter(rows_v, [jvec, cvec], t * ex)
            pltpu.sync_copy(rows_v, acc.at[didx_v], add=True)
            return carry

        lax.fori_loop(0, NCHUNK, _chunk, 0)
        plsc.subcore_barrier()

        @pl.when(sid == 0)
        def _dump():
            pltpu.sync_copy(acc.at[pl.ds(0, N_NODES)], acc_out.at[r, cid])
        for q in range(N_NODES // _BP):
            pltpu.sync_copy(den_v.at[pl.ds(q * _BP, _BP)],
                            den_out.at[r, wid, q, 0])
        plsc.subcore_barrier()


def _sc_edge(hs0, hs1, hs2, es_all, ed_all,
             src0, src1, src2, dst0, dst1, dst2):
    return pl.kernel(
        _sc_edge_body,
        out_type=[
            jax.ShapeDtypeStruct((3, 2, N_NODES, H_DIM), _f32),
            jax.ShapeDtypeStruct((3, NW, N_NODES // _BP, 1, _BP), _f32),
        ],
        mesh=_SC_MESH,
        compiler_params=pltpu.CompilerParams(needs_layout_passes=False,
                                             use_tc_tiling_on_sc=False),
        scratch_types=[
            pltpu.VMEM((N_NODES,), _f32),
            pltpu.VMEM((N_NODES,), _f32),
            pltpu.VMEM((N_NODES,), _f32),
            pltpu.VMEM((CHUNK, H_DIM), _f32),
            pltpu.VMEM((CHUNK,), _i32),
            pltpu.VMEM((CHUNK,), _i32),
            pltpu.VMEM((CHUNK, H_DIM), _f32),
            pltpu.VMEM_SHARED((ACC_ROWS, H_DIM), _f32),
            pltpu.SemaphoreType.DMA,
        ],
    )(hs0, hs1, hs2, es_all, ed_all, src0, src1, src2, dst0, dst1, dst2)


def _tc_post_body(acc_ref, den_ref, bsum_ref, wlin_ref, blin_ref, out_ref):
    acc = acc_ref[...]
    den = den_ref[...]
    o = jnp.zeros((_BP, H_DIM), _f32)
    for r in range(3):
        a = acc[r, 0] + acc[r, 1]
        d = jnp.sum(den[r], axis=0)[0, 0]
        o = o + a / (d + 1e-16)[:, None]
    p = jnp.maximum(o + bsum_ref[...][None, :], 0.0)
    out_ref[...] = jnp.dot(p, wlin_ref[...], preferred_element_type=_f32) \
        + blin_ref[...][None, :]


def _tc_post(acc, den, bsum, W_lin, b_lin):
    grid = (N_NODES // _BP,)
    return pl.pallas_call(
        _tc_post_body,
        grid=grid,
        in_specs=[
            pl.BlockSpec((3, 2, _BP, H_DIM), lambda i: (0, 0, i, 0)),
            pl.BlockSpec((3, NW, 1, 1, _BP), lambda i: (0, 0, i, 0, 0)),
            pl.BlockSpec((H_DIM,), lambda i: (0,)),
            pl.BlockSpec((H_DIM, OUT_DIM), lambda i: (0, 0)),
            pl.BlockSpec((OUT_DIM,), lambda i: (0,)),
        ],
        out_specs=pl.BlockSpec((_BP, OUT_DIM), lambda i: (i, 0)),
        out_shape=jax.ShapeDtypeStruct((N_NODES, OUT_DIM), _f32),
    )(acc, den, bsum, W_lin, b_lin)


def kernel(x_patient, x_radiomic, x_gene, edge_rp, edge_gp, edge_pp,
           W_src_rp, W_dst_rp, att_src_rp, att_dst_rp, bias_rp,
           W_src_gp, W_dst_gp, att_src_gp, att_dst_gp, bias_gp,
           W_src_pp, W_dst_pp, att_src_pp, att_dst_pp, bias_pp,
           W_lin, b_lin):
    Xs = jnp.stack([x_radiomic, x_gene, x_patient])
    Wsrc = jnp.stack([W_src_rp, W_src_gp, W_src_pp])
    Wdst = jnp.stack([W_dst_rp, W_dst_gp, W_dst_pp])
    Asrc = jnp.stack([att_src_rp, att_src_gp, att_src_pp]).reshape(3, 1, H_DIM)
    Adst = jnp.stack([att_dst_rp, att_dst_gp, att_dst_pp]).reshape(3, 1, H_DIM)
    src0 = edge_rp[0].astype(_i32)
    src1 = edge_gp[0].astype(_i32)
    src2 = edge_pp[0].astype(_i32)
    dst0 = edge_rp[1].astype(_i32)
    dst1 = edge_gp[1].astype(_i32)
    dst2 = edge_pp[1].astype(_i32)
    bsum = bias_rp + bias_gp + bias_pp

    hs_all, es_all, ed_all = _tc_pre(Xs, x_patient, Wsrc, Wdst, Asrc, Adst)
    es_all = es_all.reshape(3, 1, N_NODES)
    ed_all = ed_all.reshape(3, 1, N_NODES)
    acc, den = _sc_edge(hs_all[0], hs_all[1], hs_all[2], es_all, ed_all,
                        src0, src1, src2, dst0, dst1, dst2)
    return _tc_post(acc, den, bsum, W_lin, b_lin)
